# baseline (device time: 55884 ns/iter reference)
import jax
import jax.numpy as jnp
from jax import lax
from jax.experimental import pallas as pl
from jax.experimental.pallas import tpu as pltpu

N_DEV = 4
SEND_ORDER = (1, 3, 2)
STEP_CHUNKS = (4, 2, 2, 1)


def kernel(x, w_mat):
    m_per, k = x.shape
    _, n = w_mat.shape
    n_per = n // N_DEV
    x_rows = 256
    x_chunks = m_per // x_rows

    def body(x_hbm, w_hbm, out_ref,
             xf_ref, xbf_ref, wf_ref, wbf_ref, send_ref, recv_ref,
             xdma_sems, wdma_sems, send_sems, recv_sems):
        me = lax.axis_index("i")
        block_js = [(me + d) % N_DEV for d in SEND_ORDER] + [me]

        def x_dma(r, buf):
            return pltpu.make_async_copy(
                x_hbm.at[pl.ds(r * x_rows, x_rows), :],
                xf_ref.at[buf], xdma_sems.at[buf])

        def w_dma(j, buf):
            return pltpu.make_async_copy(
                w_hbm.at[:, pl.ds(j * n_per, n_per)],
                wf_ref.at[buf], wdma_sems.at[buf])

        def chunk_rdma(j, row0, nrows, r):
            return pltpu.make_async_remote_copy(
                src_ref=send_ref.at[j, pl.ds(row0, nrows)],
                dst_ref=recv_ref.at[me, pl.ds(row0, nrows)],
                send_sem=send_sems.at[j, r],
                recv_sem=recv_sems.at[me, r],
                device_id=(j,), device_id_type=pl.DeviceIdType.MESH)

        x_dma(0, 0).start()
        x_dma(1, 1).start()
        w_dma(block_js[0], 0).start()

        barrier_sem = pltpu.get_barrier_semaphore()
        for d in range(N_DEV):
            @pl.when(me != d)
            def _():
                pl.semaphore_signal(
                    barrier_sem, inc=1,
                    device_id=(d,), device_id_type=pl.DeviceIdType.MESH)
        pl.semaphore_wait(barrier_sem, N_DEV - 1)

        j0 = block_js[0]
        w_dma(j0, 0).wait()
        w_dma(block_js[1], 1).start()
        wbf_ref[0] = wf_ref[0].astype(jnp.bfloat16)
        for r in range(x_chunks):
            x_dma(r, r % 2).wait()
            if r + 2 < x_chunks:
                x_dma(r + 2, r % 2).start()
            rows = pl.ds(r * x_rows, x_rows)
            xbf_ref[rows, :] = xf_ref[r % 2].astype(jnp.bfloat16)
            y = jnp.dot(xbf_ref[rows, :], wbf_ref[0],
                        preferred_element_type=jnp.float32)
            send_ref[j0, rows, :] = jnp.maximum(y, 0.0).astype(jnp.bfloat16)
            chunk_rdma(j0, r * x_rows, x_rows, r).start()

        for s in (1, 2):
            j = block_js[s]
            buf = s % 2
            w_dma(j, buf).wait()
            w_dma(block_js[s + 1], (s + 1) % 2).start()
            wbf_ref[buf] = wf_ref[buf].astype(jnp.bfloat16)
            half = m_per // STEP_CHUNKS[s]
            for h in range(STEP_CHUNKS[s]):
                rows = pl.ds(h * half, half)
                y = jnp.dot(xbf_ref[rows, :], wbf_ref[buf],
                            preferred_element_type=jnp.float32)
                send_ref[j, rows, :] = (
                    jnp.maximum(y, 0.0).astype(jnp.bfloat16))
                chunk_rdma(j, h * half, half, h).start()

        w_dma(me, 1).wait()
        wbf_ref[1] = wf_ref[1].astype(jnp.bfloat16)
        y = jnp.dot(xbf_ref[...], wbf_ref[1],
                    preferred_element_type=jnp.float32)
        out_ref[pl.ds(me * m_per, m_per), :] = jnp.maximum(y, 0.0)

        for d in SEND_ORDER:
            p = (me - d) % N_DEV
            nchunks = STEP_CHUNKS[SEND_ORDER.index(d)]
            rows_per = m_per // nchunks
            for r in range(nchunks):
                recv = pltpu.make_async_remote_copy(
                    src_ref=send_ref.at[p, pl.ds(r * rows_per, rows_per)],
                    dst_ref=recv_ref.at[p, pl.ds(r * rows_per, rows_per)],
                    send_sem=send_sems.at[p, r],
                    recv_sem=recv_sems.at[p, r],
                    device_id=(p,), device_id_type=pl.DeviceIdType.MESH)
                recv.wait_recv()
                out_ref[pl.ds(p * m_per + r * rows_per, rows_per), :] = (
                    recv_ref[p, pl.ds(r * rows_per, rows_per), :]
                    .astype(jnp.float32))

        for s, d in enumerate(SEND_ORDER):
            j = (me + d) % N_DEV
            nchunks = STEP_CHUNKS[s]
            rows_per = m_per // nchunks
            for r in range(nchunks):
                send = pltpu.make_async_remote_copy(
                    src_ref=send_ref.at[j, pl.ds(r * rows_per, rows_per)],
                    dst_ref=recv_ref.at[j, pl.ds(r * rows_per, rows_per)],
                    send_sem=send_sems.at[j, r],
                    recv_sem=recv_sems.at[j, r],
                    device_id=(j,), device_id_type=pl.DeviceIdType.MESH)
                send.wait_send()

    return pl.pallas_call(
        body,
        out_shape=jax.ShapeDtypeStruct((N_DEV * m_per, n_per), jnp.float32),
        in_specs=[
            pl.BlockSpec(memory_space=pl.ANY),
            pl.BlockSpec(memory_space=pl.ANY),
        ],
        out_specs=pl.BlockSpec(memory_space=pltpu.VMEM),
        scratch_shapes=[
            pltpu.VMEM((2, x_rows, k), jnp.float32),
            pltpu.VMEM((m_per, k), jnp.bfloat16),
            pltpu.VMEM((2, k, n_per), jnp.float32),
            pltpu.VMEM((2, k, n_per), jnp.bfloat16),
            pltpu.VMEM((N_DEV, m_per, n_per), jnp.bfloat16),
            pltpu.VMEM((N_DEV, m_per, n_per), jnp.bfloat16),
            pltpu.SemaphoreType.DMA((2,)),
            pltpu.SemaphoreType.DMA((2,)),
            pltpu.SemaphoreType.DMA((N_DEV, 4)),
            pltpu.SemaphoreType.DMA((N_DEV, 4)),
        ],
        compiler_params=pltpu.CompilerParams(
            collective_id=0,
            vmem_limit_bytes=62 * 1024 * 1024,
        ),
    )(x, w_mat)
